# Initial kernel scaffold; baseline (speedup 1.0000x reference)
import jax
import jax.numpy as jnp
from jax.experimental import pallas as pl


def kernel(x, W_enc, b_enc, b_dec):
    raise NotImplementedError("write your pallas kernel here")


# speedup vs baseline: 17.9565x; 17.9565x over previous
import functools

import jax
import jax.numpy as jnp
from jax.experimental import pallas as pl
from jax.experimental.pallas import tpu as pltpu

D_IN = 768
D_HIDDEN = 8192
K = 64
N_TOK = 2048

TB_ENC = 128
TB_DEC = 256


def _encode_topk_kernel(x_ref, wt_hbm, be_ref, o_ref, wt_vmem, sem):
    @pl.when(pl.program_id(0) == 0)
    def _():
        cp = pltpu.make_async_copy(wt_hbm, wt_vmem, sem)
        cp.start()
        cp.wait()

    enc = jnp.dot(x_ref[...], wt_vmem[...], preferred_element_type=jnp.float32)
    s = jnp.maximum(enc + be_ref[...], 0.0)
    si = jax.lax.bitcast_convert_type(s, jnp.int32)

    lo0 = jnp.zeros((TB_ENC, 1), jnp.int32)
    hi0 = jnp.full((TB_ENC, 1), jnp.int32(0x7F800000))

    def body(_, carry):
        lo, hi = carry
        mid = lo + (hi - lo) // 2
        cnt = jnp.sum((si >= mid).astype(jnp.int32), axis=1, keepdims=True)
        pred = cnt >= K
        return jnp.where(pred, mid, lo), jnp.where(pred, hi, mid)

    lo, _ = jax.lax.fori_loop(0, 31, body, (lo0, hi0))
    o_ref[...] = jnp.where(si >= lo, s, 0.0)


def _decode_kernel(s_ref, w_hbm, bd_ref, o_ref, w_vmem, sem):
    @pl.when(pl.program_id(0) == 0)
    def _():
        cp = pltpu.make_async_copy(w_hbm, w_vmem, sem)
        cp.start()
        cp.wait()

    o_ref[...] = (
        jnp.dot(s_ref[...], w_vmem[...], preferred_element_type=jnp.float32)
        + bd_ref[...]
    )


@jax.jit
def kernel(x, W_enc, b_enc, b_dec):
    n = x.shape[0]
    wt = W_enc.T

    sparse = pl.pallas_call(
        _encode_topk_kernel,
        grid=(n // TB_ENC,),
        in_specs=[
            pl.BlockSpec((TB_ENC, D_IN), lambda i: (i, 0)),
            pl.BlockSpec(memory_space=pl.ANY),
            pl.BlockSpec((1, D_HIDDEN), lambda i: (0, 0)),
        ],
        out_specs=pl.BlockSpec((TB_ENC, D_HIDDEN), lambda i: (i, 0)),
        out_shape=jax.ShapeDtypeStruct((n, D_HIDDEN), jnp.float32),
        scratch_shapes=[
            pltpu.VMEM((D_IN, D_HIDDEN), jnp.float32),
            pltpu.SemaphoreType.DMA,
        ],
    )(x, wt, b_enc.reshape(1, D_HIDDEN))

    out = pl.pallas_call(
        _decode_kernel,
        grid=(n // TB_DEC,),
        in_specs=[
            pl.BlockSpec((TB_DEC, D_HIDDEN), lambda i: (i, 0)),
            pl.BlockSpec(memory_space=pl.ANY),
            pl.BlockSpec((1, D_IN), lambda i: (0, 0)),
        ],
        out_specs=pl.BlockSpec((TB_DEC, D_IN), lambda i: (i, 0)),
        out_shape=jax.ShapeDtypeStruct((n, D_IN), jnp.float32),
        scratch_shapes=[
            pltpu.VMEM((D_HIDDEN, D_IN), jnp.float32),
            pltpu.SemaphoreType.DMA,
        ],
    )(sparse, W_enc, b_dec.reshape(1, D_IN))

    return out
